# Initial kernel scaffold; baseline (speedup 1.0000x reference)
import jax
import jax.numpy as jnp
from jax.experimental import pallas as pl


def kernel(log_factor_potentials, factor_var_idx, alphas):
    raise NotImplementedError("write your pallas kernel here")


# speedup vs baseline: 45.0366x; 45.0366x over previous
import functools

import jax
import jax.numpy as jnp
from jax import lax
from jax.experimental import pallas as pl
from jax.experimental.pallas import tpu as pltpu
from jax.experimental.pallas import tpu_sc as plsc

F = 800_000
V = 100_000
ITERS = 5

NC = 2
NS = 16
LN = 16

FPAD = 802_816
VPAD = 100_352
ET = 2 * FPAD // NS
VS = VPAD // NS
CK = 6272
RW = 512
R = FPAD // RW
BR = 112

_mesh = plsc.VectorSubcoreMesh(
    core_axis_name="c", subcore_axis_name="s", num_cores=NC, num_subcores=NS)
_sc_params = pltpu.CompilerParams(needs_layout_passes=False)

_f32 = jnp.float32


def _zero_vmem(ref, n):
    def zb(i, _):
        ref[pl.ds(i * LN, LN)] = jnp.zeros((LN,), _f32)
        return ()
    lax.fori_loop(0, n // LN, zb, (), unroll=8)


@functools.partial(
    pl.kernel,
    out_type=(
        jax.ShapeDtypeStruct((NC, VPAD), _f32),
        jax.ShapeDtypeStruct((NC, NS, VPAD), _f32),
    ),
    mesh=_mesh,
    compiler_params=_sc_params,
    scratch_types=[
        pltpu.VMEM((VPAD,), _f32),
        pltpu.VMEM((CK,), jnp.int32),
        pltpu.VMEM((CK,), _f32),
        pltpu.VMEM((VS,), _f32),
        pltpu.VMEM((VS,), _f32),
    ],
)
def _scatter(idx_hbm, val_hbm, vb_hbm, scr_hbm, table, idxb, valb, ubuf, accb):
    c = lax.axis_index("c")
    t = lax.axis_index("s")
    s_t = t // 8
    fbase = (t % 8) * ET
    plane = s_t * 2 + c
    _zero_vmem(table, VPAD)

    def chunk(kc, _):
        off = fbase + kc * CK
        pltpu.sync_copy(idx_hbm.at[s_t, pl.ds(off, CK)], idxb)
        pltpu.sync_copy(val_hbm.at[plane, pl.ds(off, CK)], valb)

        def inner(i, _):
            iv = idxb[pl.ds(i * LN, LN)]
            vv = valb[pl.ds(i * LN, LN)]
            plsc.addupdate_scatter(table, [iv], vv)
            return ()
        lax.fori_loop(0, CK // LN, inner, (), unroll=4)
        return ()
    lax.fori_loop(0, ET // CK, chunk, ())

    pltpu.sync_copy(table, scr_hbm.at[c, t])
    plsc.subcore_barrier()

    vo = t * VS
    _zero_vmem(accb, VS)

    def mrg(u, _):
        pltpu.sync_copy(scr_hbm.at[c, u, pl.ds(vo, VS)], ubuf)

        def addv(i, _):
            accb[pl.ds(i * LN, LN)] = (
                accb[pl.ds(i * LN, LN)] + ubuf[pl.ds(i * LN, LN)])
            return ()
        lax.fori_loop(0, VS // LN, addv, (), unroll=8)
        return ()
    lax.fori_loop(0, NS, mrg, ())
    pltpu.sync_copy(accb, vb_hbm.at[c, pl.ds(vo, VS)])


@functools.partial(
    pl.kernel,
    out_type=jax.ShapeDtypeStruct((4, FPAD), _f32),
    mesh=_mesh,
    compiler_params=_sc_params,
    scratch_types=[
        pltpu.VMEM((VPAD,), _f32),
        pltpu.VMEM((CK,), jnp.int32),
        pltpu.VMEM((CK,), _f32),
    ],
)
def _gather(vb_hbm, idx_hbm, g_hbm, table, idxb, gbuf):
    c = lax.axis_index("c")
    t = lax.axis_index("s")
    s_t = t // 8
    fbase = (t % 8) * ET
    plane = s_t * 2 + c
    pltpu.sync_copy(vb_hbm.at[c], table)

    def chunk(kc, _):
        off = fbase + kc * CK
        pltpu.sync_copy(idx_hbm.at[s_t, pl.ds(off, CK)], idxb)

        def inner(i, _):
            iv = idxb[pl.ds(i * LN, LN)]
            gbuf[pl.ds(i * LN, LN)] = plsc.load_gather(table, [iv])
            return ()
        lax.fori_loop(0, CK // LN, inner, (), unroll=4)
        pltpu.sync_copy(gbuf, g_hbm.at[plane, pl.ds(off, CK)])
        return ()
    lax.fori_loop(0, ET // CK, chunk, ())


def _lse(a, b):
    m = jnp.maximum(a, b)
    return m + jnp.log1p(jnp.exp(-jnp.abs(a - b)))


def _p3_body(a_ref, g_ref, f_ref, v_ref, lp_ref, of_ref, ov_ref):
    a = a_ref[0, 0]
    g = g_ref[...]
    f_ = f_ref[...]
    v_ = v_ref[...]
    lp = lp_ref[...]
    nv = g - f_
    l0 = _lse(nv[0], nv[1])
    l1 = _lse(nv[2], nv[3])
    v00 = a * (nv[0] - l0) + (1 - a) * v_[0]
    v01 = a * (nv[1] - l0) + (1 - a) * v_[1]
    v10 = a * (nv[2] - l1) + (1 - a) * v_[2]
    v11 = a * (nv[3] - l1) + (1 - a) * v_[3]
    t00 = _lse(lp[0] + v10, lp[1] + v11)
    t01 = _lse(lp[2] + v10, lp[3] + v11)
    t10 = _lse(lp[0] + v00, lp[2] + v01)
    t11 = _lse(lp[1] + v00, lp[3] + v01)
    n0 = _lse(t00, t01)
    n1 = _lse(t10, t11)
    b = 1 - a
    of_ref[...] = jnp.stack([
        a * (t00 - n0) + b * f_[0],
        a * (t01 - n0) + b * f_[1],
        a * (t10 - n1) + b * f_[2],
        a * (t11 - n1) + b * f_[3],
    ])
    ov_ref[...] = jnp.stack([v00, v01, v10, v11])


def _p3(alpha, g4, f4, v4, lp4):
    blk = pl.BlockSpec((4, BR, RW), lambda r: (0, r, 0))
    return pl.pallas_call(
        _p3_body,
        grid=(R // BR,),
        in_specs=[
            pl.BlockSpec(memory_space=pltpu.SMEM),
            blk, blk, blk, blk,
        ],
        out_specs=[blk, blk],
        out_shape=(
            jax.ShapeDtypeStruct((4, R, RW), _f32),
            jax.ShapeDtypeStruct((4, R, RW), _f32),
        ),
    )(alpha, g4, f4, v4, lp4)


def _cf_body(lp_ref, v_ref, out_ref):
    r = pl.program_id(0)

    @pl.when(r == 0)
    def _():
        out_ref[...] = jnp.zeros_like(out_ref)

    lp = lp_ref[...]
    v_ = v_ref[...]
    fb0 = lp[0] + v_[0] + v_[2]
    fb1 = lp[1] + v_[0] + v_[3]
    fb2 = lp[2] + v_[1] + v_[2]
    fb3 = lp[3] + v_[1] + v_[3]
    m4 = jnp.maximum(jnp.maximum(fb0, fb1), jnp.maximum(fb2, fb3))
    e0 = jnp.exp(fb0 - m4)
    e1 = jnp.exp(fb1 - m4)
    e2 = jnp.exp(fb2 - m4)
    e3 = jnp.exp(fb3 - m4)
    s = e0 + e1 + e2 + e3
    ls = m4 + jnp.log(s)
    contrib = (e0 * (fb0 - ls - lp[0]) + e1 * (fb1 - ls - lp[1])
               + e2 * (fb2 - ls - lp[2]) + e3 * (fb3 - ls - lp[3])) / s
    rows = lax.broadcasted_iota(jnp.int32, (BR, RW), 0)
    cols = lax.broadcasted_iota(jnp.int32, (BR, RW), 1)
    gidx = (r * BR + rows) * RW + cols
    contrib = jnp.where(gidx < F, contrib, 0.0)
    red = jnp.sum(contrib, axis=0, keepdims=True)
    rid = lax.broadcasted_iota(jnp.int32, (8, RW), 0)
    out_ref[...] += jnp.where(rid == 0, red, 0.0)


def _cf(lp4, v4):
    blk = pl.BlockSpec((4, BR, RW), lambda r: (0, r, 0))
    return pl.pallas_call(
        _cf_body,
        grid=(R // BR,),
        in_specs=[blk, blk],
        out_specs=pl.BlockSpec((8, RW), lambda r: (0, 0)),
        out_shape=jax.ShapeDtypeStruct((8, RW), _f32),
    )(lp4, v4)


def _cv_body(vb_ref, deg_ref, out_ref):
    vb = vb_ref[...]
    deg = deg_ref[...]
    l = _lse(vb[0], vb[1])
    nb0 = vb[0] - l
    nb1 = vb[1] - l
    contrib = (1.0 - deg) * (jnp.exp(nb0) * nb0 + jnp.exp(nb1) * nb1)
    rows = lax.broadcasted_iota(jnp.int32, (VPAD // RW, RW), 0)
    cols = lax.broadcasted_iota(jnp.int32, (VPAD // RW, RW), 1)
    gidx = rows * RW + cols
    contrib = jnp.where(gidx < V, contrib, 0.0)
    red = jnp.sum(contrib, axis=0, keepdims=True)
    rid = lax.broadcasted_iota(jnp.int32, (8, RW), 0)
    out_ref[...] = jnp.where(rid == 0, red, 0.0)


def _cv(vb2, deg2):
    return pl.pallas_call(
        _cv_body,
        out_shape=jax.ShapeDtypeStruct((8, RW), _f32),
    )(vb2, deg2)


def kernel(log_factor_potentials, factor_var_idx, alphas):
    lpT = jnp.pad(log_factor_potentials.reshape(F, 4).T,
                  ((0, 0), (0, FPAD - F)))
    idxT = jnp.pad(factor_var_idx.T.astype(jnp.int32),
                   ((0, 0), (0, FPAD - F)), constant_values=V)
    lp4 = lpT.reshape(4, R, RW)

    f4 = jnp.zeros((4, R, RW), _f32)
    v4 = jnp.zeros((4, R, RW), _f32)
    for i in range(ITERS):
        if i == 0:
            g4 = jnp.zeros((4, R, RW), _f32)
        else:
            vb, _ = _scatter(idxT, f4.reshape(4, FPAD))
            g4 = _gather(vb, idxT).reshape(4, R, RW)
        a2 = alphas[i].reshape(1, 1)
        f4, v4 = _p3(a2, g4, f4, v4, lp4)

    vbf, _ = _scatter(idxT, f4.reshape(4, FPAD))
    ones = jnp.zeros((4, FPAD), _f32).at[:, :F].set(1.0)
    degb, _ = _scatter(idxT, ones)

    cf = _cf(lp4, v4)
    cv = _cv(vbf.reshape(NC, VPAD // RW, RW), degb[0].reshape(VPAD // RW, RW))
    return -(jnp.sum(cf) + jnp.sum(cv))
